# baseline (speedup 1.0000x reference)
import functools

import jax
import jax.numpy as jnp
from jax import lax
from jax.experimental import pallas as pl
from jax.experimental.pallas import tpu as pltpu
from jax.experimental.pallas import tpu_sc as plsc

B, N1, N2, D = 2, 10000, 50000, 128
E_PIPE, E_SURF, E_C = 160000, 600000, 40000

NC, NS, L = 2, 16, 16
SEG = 2048
G = 64
CE = 4096
CBLK = CE // G
NP1 = 10240
NCH = 10240
NSC = 5
NSURF = NSC * NCH
ACC_ROWS = 10496
ZPT = ACC_ROWS // NS
TRASH = NCH
SEGS_PIPE, SEGS_SURF, SEGS_C = 5, 19, 2
RTC = 512


def _chunks(total, step):
    out, off = [], 0
    while off < total:
        sz = min(step, total - off)
        out.append((off, sz))
        off += sz
    return out


def _sc_segment_sums(z1f, z2f, eip, eis, e12, e21, zvec_h, ones_h):
    mesh = plsc.VectorSubcoreMesh(core_axis_name="c", subcore_axis_name="s",
                                  num_cores=NC, num_subcores=NS)
    f32, i32 = jnp.float32, jnp.int32
    out_type = [
        jax.ShapeDtypeStruct((B * NP1, D), f32),
        jax.ShapeDtypeStruct((B * NP1,), f32),
        jax.ShapeDtypeStruct((B * NSURF, D), f32),
        jax.ShapeDtypeStruct((B * NSURF,), f32),
        jax.ShapeDtypeStruct((B * NP1, D), f32),
        jax.ShapeDtypeStruct((B * NP1,), f32),
        jax.ShapeDtypeStruct((B * NP1, D), f32),
        jax.ShapeDtypeStruct((B * NP1,), f32),
    ]
    scratch = [
        pltpu.VMEM_SHARED((ACC_ROWS, D), f32),
        pltpu.VMEM_SHARED((ACC_ROWS,), f32),
        pltpu.VMEM((SEG,), i32),
        pltpu.VMEM((SEG,), i32),
        pltpu.VMEM((SEG,), i32),
        pltpu.VMEM((SEG,), i32),
        pltpu.VMEM((CE,), i32),
        pltpu.VMEM((CBLK, G), i32),
        pltpu.VMEM((G, D), f32),
        pltpu.VMEM((G, D), f32),
        pltpu.VMEM((ZPT,), f32),
        pltpu.VMEM((G,), f32),
        pltpu.VMEM((NCH // NS,), f32),
        pltpu.VMEM((NCH // NS,), f32),
        pltpu.SemaphoreType.DMA,
        pltpu.SemaphoreType.DMA,
        pltpu.SemaphoreType.DMA,
        pltpu.SemaphoreType.DMA,
        pltpu.SemaphoreType.DMA,
        pltpu.SemaphoreType.DMA,
    ]

    @functools.partial(
        pl.kernel, out_type=out_type, mesh=mesh, scratch_types=scratch,
        compiler_params=pltpu.CompilerParams(needs_layout_passes=False))
    def body(z1_h, z2_h, ep_h, es_h, e12_h, e21_h, zv_h, on_h,
             sum_p, inv_p, sum_s, inv_s, sum_12, inv_12, sum_21, inv_21,
             acc_sum, acc_cnt, seg_srcA, seg_dstA, seg_srcB, seg_dstB,
             csrc, cdst, rowsA, rowsB,
             zvec, onesv, cvec, ivec, gsA, gsB, ssA, ssB, stA, stB):
        b = lax.axis_index("c")
        sid = lax.axis_index("s")
        iota = lax.broadcasted_iota(i32, (L,), 0)
        pltpu.sync_copy(zv_h, zvec)
        pltpu.sync_copy(on_h, onesv)

        def zero_rows():
            def zb(r, _):
                for c in range(D // L):
                    rowsA[r, pl.ds(c * L, L)] = jnp.zeros((L,), f32)
                return 0
            lax.fori_loop(0, G, zb, 0)

        def with_parity(even, fa, fb):
            @pl.when(even)
            def _():
                fa()
            @pl.when(jnp.logical_not(even))
            def _():
                fb()

        def run_task(e_h, E, segs, z_h, nsrc, np_t, lo, out_sum, out_inv,
                     out_base):
            zero_rows()
            zb = sid * ZPT
            for off, sz in _chunks(ZPT, G):
                pltpu.sync_copy(rowsA.at[pl.ds(0, sz)],
                                acc_sum.at[pl.ds(zb + off, sz)])
            pltpu.sync_copy(zvec, acc_cnt.at[pl.ds(zb, ZPT)])
            plsc.subcore_barrier()

            def g_issue(j, rows_p, gs_p):
                slot = lax.bitwise_and(j, CBLK - 1)
                pltpu.async_copy(z_h.at[csrc.at[pl.ds(slot * G, G)]],
                                 rows_p, gs_p)

            def g_wait(rows_p, gs_p):
                pltpu.make_async_copy(z_h.at[csrc.at[pl.ds(0, G)]],
                                      rows_p, gs_p).wait()

            def s_issue(j, rows_p, ss_p):
                slot = lax.bitwise_and(j, CBLK - 1)
                drow = cdst.at[slot]
                pltpu.async_copy(rows_p, acc_sum.at[drow], ss_p, add=True)
                pltpu.async_copy(onesv, acc_cnt.at[drow], ss_p, add=True)

            def s_wait(rows_p, ss_p):
                pltpu.make_async_copy(rows_p, acc_sum.at[cdst.at[0]],
                                      ss_p).wait()
                pltpu.make_async_copy(onesv, acc_cnt.at[cdst.at[0]],
                                      ss_p).wait()

            def fbody(j, _):
                even = lax.bitwise_and(j, 1) == 0
                @pl.when(j >= 2)
                def _():
                    with_parity(even, lambda: s_wait(rowsA, ssA),
                                lambda: s_wait(rowsB, ssB))
                with_parity(even, lambda: g_issue(j, rowsA, gsA),
                            lambda: g_issue(j, rowsB, gsB))
                @pl.when(j >= 1)
                def _():
                    def prevA():
                        g_wait(rowsA, gsA)
                        s_issue(j - 1, rowsA, ssA)
                    def prevB():
                        g_wait(rowsB, gsB)
                        s_issue(j - 1, rowsB, ssB)
                    with_parity(jnp.logical_not(even), prevA, prevB)
                return 0

            stripe = segs * SEG
            base_e = sid * stripe
            hi = lo + np_t

            def st_issue(sgi, ssrc, sdst, st_p):
                sbase = base_e + sgi * SEG
                pltpu.async_copy(e_h.at[0, pl.ds(sbase, SEG)], ssrc, st_p)
                pltpu.async_copy(e_h.at[1, pl.ds(sbase, SEG)], sdst, st_p)

            def st_wait(ssrc, sdst, st_p):
                pltpu.make_async_copy(e_h.at[0, pl.ds(0, SEG)], ssrc,
                                      st_p).wait()
                pltpu.make_async_copy(e_h.at[1, pl.ds(0, SEG)], sdst,
                                      st_p).wait()

            st_issue(0, seg_srcA, seg_dstA, stA)

            def seg_body(sgi, carry):
                pos0, bfl = carry
                sbase = base_e + sgi * SEG
                seven = lax.bitwise_and(sgi, 1) == 0
                with_parity(seven,
                            lambda: st_wait(seg_srcA, seg_dstA, stA),
                            lambda: st_wait(seg_srcB, seg_dstB, stB))

                @pl.when(sgi + 1 < segs)
                def _():
                    with_parity(
                        seven,
                        lambda: st_issue(sgi + 1, seg_srcB, seg_dstB, stB),
                        lambda: st_issue(sgi + 1, seg_srcA, seg_dstA, stA))

                def cbody(i, off):
                    s = jnp.where(seven, seg_srcA[pl.ds(i * L, L)],
                                  seg_srcB[pl.ds(i * L, L)])
                    d = jnp.where(seven, seg_dstA[pl.ds(i * L, L)],
                                  seg_dstB[pl.ds(i * L, L)])
                    geid = sbase + i * L + iota
                    m = (geid < E) & (d >= lo) & (d < hi)
                    mi = jnp.where(m, 1, 0).astype(i32)
                    pos = off + plsc.cumsum(mi) - 1
                    posr = lax.bitwise_and(pos, CE - 1)
                    plsc.store_scatter(csrc, [posr], s + b * nsrc, mask=m)
                    plsc.store_scatter(
                        cdst,
                        [lax.shift_right_logical(posr, 6),
                         lax.bitwise_and(posr, G - 1)],
                        d - lo, mask=m)
                    return off + plsc.all_reduce_population_count(m)

                pos1v = lax.fori_loop(0, SEG // L, cbody, pos0, unroll=2)
                pos1 = jnp.max(pos1v)
                bhi = lax.shift_right_logical(pos1, 6)
                lax.fori_loop(bfl, bhi, fbody, 0)
                return pos1v, bhi

            posv, bfl = lax.fori_loop(
                0, segs, seg_body, (jnp.zeros((L,), i32), jnp.int32(0)))
            pos = jnp.max(posv)

            nblk = lax.shift_right_logical(pos + (G - 1), 6)
            lastb = (nblk - 1) * G

            def nb(i, _):
                posv = lastb + i * L + iota
                mm = posv >= pos
                posr = lax.bitwise_and(posv, CE - 1)
                plsc.store_scatter(csrc, [posr], iota, mask=mm)
                plsc.store_scatter(
                    cdst,
                    [lax.shift_right_logical(posr, 6),
                     lax.bitwise_and(posr, G - 1)],
                    TRASH + iota, mask=mm)
                return 0

            lax.fori_loop(0, G // L, nb, 0)
            lax.fori_loop(bfl, nblk, fbody, 0)
            lastev = lax.bitwise_and(nblk - 1, 1) == 0

            @pl.when(nblk >= 1)
            def _():
                def lastA():
                    g_wait(rowsA, gsA)
                    s_issue(nblk - 1, rowsA, ssA)
                def lastB():
                    g_wait(rowsB, gsB)
                    s_issue(nblk - 1, rowsB, ssB)
                with_parity(lastev, lastA, lastB)

            @pl.when(nblk >= 2)
            def _():
                with_parity(jnp.logical_not(lastev),
                            lambda: s_wait(rowsA, ssA),
                            lambda: s_wait(rowsB, ssB))

            @pl.when(nblk >= 1)
            def _():
                with_parity(lastev, lambda: s_wait(rowsA, ssA),
                            lambda: s_wait(rowsB, ssB))

            plsc.subcore_barrier()

            rpt = np_t // NS
            rb = sid * rpt
            for off, sz in _chunks(rpt, G):
                pltpu.sync_copy(acc_sum.at[pl.ds(rb + off, sz)],
                                rowsA.at[pl.ds(0, sz)])
                pltpu.sync_copy(rowsA.at[pl.ds(0, sz)],
                                out_sum.at[pl.ds(out_base + rb + off, sz)])
            pltpu.sync_copy(acc_cnt.at[pl.ds(rb, rpt)], cvec.at[pl.ds(0, rpt)])

            def ib(i, _):
                c = cvec[pl.ds(i * L, L)]
                ivec[pl.ds(i * L, L)] = 1.0 / jnp.maximum(c, 1.0)
                return 0

            lax.fori_loop(0, rpt // L, ib, 0)
            pltpu.sync_copy(ivec.at[pl.ds(0, rpt)],
                            out_inv.at[pl.ds(out_base + rb, rpt)])
            plsc.subcore_barrier()

        run_task(ep_h, E_PIPE, SEGS_PIPE, z1_h, N1, NP1, 0, sum_p, inv_p,
                 b * NP1)
        run_task(e21_h, E_C, SEGS_C, z2_h, N2, NP1, 0, sum_21, inv_21,
                 b * NP1)
        run_task(e12_h, E_C, SEGS_C, z1_h, N1, NP1, 0, sum_12, inv_12,
                 b * NP1)

        def surf_chunk(k, _):
            run_task(es_h, E_SURF, SEGS_SURF, z2_h, N2, NCH, k * NCH,
                     sum_s, inv_s, b * NSURF + k * NCH)
            return 0

        lax.fori_loop(0, NSC, surf_chunk, 0)

    return body(z1f, z2f, eip, eis, e12, e21, zvec_h, ones_h)


def _tc_head(sum_a, inv_a, sum_b, inv_b, z, wla, wlb, wra, wrb, wp,
             bla, blb, bp, g, bln, nb_b):
    Bz, NZ, _ = z.shape
    NB = (NZ + RTC - 1) // RTC
    f32 = jnp.float32

    def body(sa, ia, sb, ib_, zz, rwla, rwlb, rwra, rwrb, rwp, rba, rbb,
             rbp, rg, rbl, o):
        i = pl.program_id(1)
        agg_a = sa[0] * ia[0, 0, :][:, None]
        agg_b = sb[0] * ib_[0, 0, :][:, None]
        agg_b = jnp.where(i < nb_b, agg_b, 0.0)
        h = (jnp.dot(agg_a, rwla[...], preferred_element_type=f32)
             + jnp.dot(agg_b, rwlb[...], preferred_element_type=f32)
             + jnp.dot(zz[0], rwra[...] + rwrb[...],
                       preferred_element_type=f32)
             + rba[0] + rbb[0])
        y = jnp.dot(h, rwp[...], preferred_element_type=f32) + rbp[0]
        m = jnp.mean(y, axis=-1, keepdims=True)
        yc = y - m
        v = jnp.mean(yc * yc, axis=-1, keepdims=True)
        o[0] = yc * lax.rsqrt(v + 1e-5) * rg[0] + rbl[0]

    in_specs = [
        pl.BlockSpec((1, RTC, D), lambda b, i: (b, i, 0)),
        pl.BlockSpec((1, 1, RTC), lambda b, i: (b, 0, i)),
        pl.BlockSpec((1, RTC, D), lambda b, i: (b, jnp.minimum(i, nb_b - 1), 0)),
        pl.BlockSpec((1, 1, RTC), lambda b, i: (b, 0, jnp.minimum(i, nb_b - 1))),
        pl.BlockSpec((1, RTC, D), lambda b, i: (b, i, 0)),
    ] + [pl.BlockSpec((D, D), lambda b, i: (0, 0))] * 5 \
      + [pl.BlockSpec((1, D), lambda b, i: (0, 0))] * 5
    return pl.pallas_call(
        body,
        grid=(Bz, NB),
        in_specs=in_specs,
        out_specs=pl.BlockSpec((1, RTC, D), lambda b, i: (b, i, 0)),
        out_shape=jax.ShapeDtypeStruct((Bz, NZ, D), f32),
    )(sum_a, inv_a, sum_b, inv_b, z, wla, wlb, wra, wrb, wp,
      bla, blb, bp, g, bln)


def kernel(z_1d, z_2d, edge_index_pipe, edge_index_surface, edge_index_c12,
           edge_index_c21, params):
    f32, i32 = jnp.float32, jnp.int32
    z1f = z_1d.reshape(B * N1, D)
    z2f = z_2d.reshape(B * N2, D)

    def prep(ei, segs):
        pe = NS * segs * SEG
        ei = ei.astype(i32)
        return jnp.pad(ei, ((0, 0), (0, pe - ei.shape[1])))

    eip = prep(edge_index_pipe, SEGS_PIPE)
    eis = prep(edge_index_surface, SEGS_SURF)
    e12 = prep(edge_index_c12, SEGS_C)
    e21 = prep(edge_index_c21, SEGS_C)
    zvec_h = jnp.zeros((ZPT,), f32)
    ones_h = jnp.ones((G,), f32)

    (sum_p, inv_p, sum_s, inv_s, sum_12, inv_12, sum_21, inv_21) = \
        _sc_segment_sums(z1f, z2f, eip, eis, e12, e21, zvec_h, ones_h)

    sum_p = sum_p.reshape(B, NP1, D)
    sum_s = sum_s.reshape(B, NSURF, D)
    sum_12 = sum_12.reshape(B, NP1, D)
    sum_21 = sum_21.reshape(B, NP1, D)
    inv_p = inv_p.reshape(B, 1, NP1)
    inv_s = inv_s.reshape(B, 1, NSURF)
    inv_12 = inv_12.reshape(B, 1, NP1)
    inv_21 = inv_21.reshape(B, 1, NP1)

    p = params
    r2 = lambda v: v.reshape(1, D)

    d1 = _tc_head(sum_p, inv_p, sum_21, inv_21, z_1d,
                  p['pipe']['Wl'], p['c21']['Wl'],
                  p['pipe']['Wr'], p['c21']['Wr'], p['proj_1d']['W'],
                  r2(p['pipe']['bl']), r2(p['c21']['bl']),
                  r2(p['proj_1d']['b']), r2(p['ln_1d']['g']),
                  r2(p['ln_1d']['b']), NP1 // RTC)
    d2 = _tc_head(sum_s, inv_s, sum_12, inv_12, z_2d,
                  p['surface']['Wl'], p['c12']['Wl'],
                  p['surface']['Wr'], p['c12']['Wr'], p['proj_2d']['W'],
                  r2(p['surface']['bl']), r2(p['c12']['bl']),
                  r2(p['proj_2d']['b']), r2(p['ln_2d']['g']),
                  r2(p['ln_2d']['b']), NP1 // RTC)
    return d1, d2


# speedup vs baseline: 1.0109x; 1.0109x over previous
import functools

import jax
import jax.numpy as jnp
from jax import lax
from jax.experimental import pallas as pl
from jax.experimental.pallas import tpu as pltpu
from jax.experimental.pallas import tpu_sc as plsc

B, N1, N2, D = 2, 10000, 50000, 128
E_PIPE, E_SURF, E_C = 160000, 600000, 40000

NC, NS, L = 2, 16, 16
SEG = 2048
G = 64
CE = 4096
CBLK = CE // G
NP1 = 10240
NCH = 10240
NSC = 5
NSURF = NSC * NCH
ACC_ROWS = 10496
ZPT = ACC_ROWS // NS
TRASH = NCH
SEGS_PIPE, SEGS_SURF, SEGS_C = 5, 19, 2
RTC = 512


def _chunks(total, step):
    out, off = [], 0
    while off < total:
        sz = min(step, total - off)
        out.append((off, sz))
        off += sz
    return out


def _sc_segment_sums(z1f, z2f, eip, eis, e12, e21, zvec_h, ones_h):
    mesh = plsc.VectorSubcoreMesh(core_axis_name="c", subcore_axis_name="s",
                                  num_cores=NC, num_subcores=NS)
    f32, i32 = jnp.float32, jnp.int32
    out_type = [
        jax.ShapeDtypeStruct((B * NP1, D), f32),
        jax.ShapeDtypeStruct((B * NP1,), f32),
        jax.ShapeDtypeStruct((B * NSURF, D), f32),
        jax.ShapeDtypeStruct((B * NSURF,), f32),
        jax.ShapeDtypeStruct((B * NP1, D), f32),
        jax.ShapeDtypeStruct((B * NP1,), f32),
        jax.ShapeDtypeStruct((B * NP1, D), f32),
        jax.ShapeDtypeStruct((B * NP1,), f32),
    ]
    scratch = [
        pltpu.VMEM_SHARED((ACC_ROWS, D), f32),
        pltpu.VMEM_SHARED((ACC_ROWS,), f32),
        pltpu.VMEM((SEG,), i32),
        pltpu.VMEM((SEG,), i32),
        pltpu.VMEM((SEG,), i32),
        pltpu.VMEM((SEG,), i32),
        pltpu.VMEM((CE,), i32),
        pltpu.VMEM((CBLK, G), i32),
        pltpu.VMEM((G, D), f32),
        pltpu.VMEM((G, D), f32),
        pltpu.VMEM((ZPT,), f32),
        pltpu.VMEM((G,), f32),
        pltpu.VMEM((NCH // NS,), f32),
        pltpu.VMEM((NCH // NS,), f32),
        pltpu.SemaphoreType.DMA,
        pltpu.SemaphoreType.DMA,
        pltpu.SemaphoreType.DMA,
        pltpu.SemaphoreType.DMA,
        pltpu.SemaphoreType.DMA,
        pltpu.SemaphoreType.DMA,
        pltpu.SemaphoreType.DMA,
    ]

    @functools.partial(
        pl.kernel, out_type=out_type, mesh=mesh, scratch_types=scratch,
        compiler_params=pltpu.CompilerParams(needs_layout_passes=False))
    def body(z1_h, z2_h, ep_h, es_h, e12_h, e21_h, zv_h, on_h,
             sum_p, inv_p, sum_s, inv_s, sum_12, inv_12, sum_21, inv_21,
             acc_sum, acc_cnt, seg_srcA, seg_dstA, seg_srcB, seg_dstB,
             csrc, cdst, rowsA, rowsB,
             zvec, onesv, cvec, ivec, gsA, gsB, ssA, ssB, stA, stB, wsem):
        b = lax.axis_index("c")
        sid = lax.axis_index("s")
        iota = lax.broadcasted_iota(i32, (L,), 0)
        pltpu.sync_copy(zv_h, zvec)
        pltpu.sync_copy(on_h, onesv)

        def zero_rows():
            def zb(r, _):
                for c in range(D // L):
                    rowsA[r, pl.ds(c * L, L)] = jnp.zeros((L,), f32)
                return 0
            lax.fori_loop(0, G, zb, 0)

        def with_parity(even, fa, fb):
            @pl.when(even)
            def _():
                fa()
            @pl.when(jnp.logical_not(even))
            def _():
                fb()

        def run_task(e_h, E, segs, z_h, nsrc, np_t, lo, out_sum, out_inv,
                     out_base):
            zero_rows()
            zb = sid * ZPT
            zchunks = _chunks(ZPT, G)
            for off, sz in zchunks:
                pltpu.async_copy(rowsA.at[pl.ds(0, sz)],
                                 acc_sum.at[pl.ds(zb + off, sz)], wsem)
            pltpu.async_copy(zvec, acc_cnt.at[pl.ds(zb, ZPT)], wsem)
            for off, sz in zchunks:
                pltpu.make_async_copy(rowsA.at[pl.ds(0, sz)],
                                      acc_sum.at[pl.ds(zb + off, sz)],
                                      wsem).wait()
            pltpu.make_async_copy(zvec, acc_cnt.at[pl.ds(zb, ZPT)],
                                  wsem).wait()
            plsc.subcore_barrier()

            def g_issue(j, rows_p, gs_p):
                slot = lax.bitwise_and(j, CBLK - 1)
                pltpu.async_copy(z_h.at[csrc.at[pl.ds(slot * G, G)]],
                                 rows_p, gs_p)

            def g_wait(rows_p, gs_p):
                pltpu.make_async_copy(z_h.at[csrc.at[pl.ds(0, G)]],
                                      rows_p, gs_p).wait()

            def s_issue(j, rows_p, ss_p):
                slot = lax.bitwise_and(j, CBLK - 1)
                drow = cdst.at[slot]
                pltpu.async_copy(rows_p, acc_sum.at[drow], ss_p, add=True)
                pltpu.async_copy(onesv, acc_cnt.at[drow], ss_p, add=True)

            def s_wait(rows_p, ss_p):
                pltpu.make_async_copy(rows_p, acc_sum.at[cdst.at[0]],
                                      ss_p).wait()
                pltpu.make_async_copy(onesv, acc_cnt.at[cdst.at[0]],
                                      ss_p).wait()

            def fbody(j, _):
                even = lax.bitwise_and(j, 1) == 0
                @pl.when(j >= 2)
                def _():
                    with_parity(even, lambda: s_wait(rowsA, ssA),
                                lambda: s_wait(rowsB, ssB))
                with_parity(even, lambda: g_issue(j, rowsA, gsA),
                            lambda: g_issue(j, rowsB, gsB))
                @pl.when(j >= 1)
                def _():
                    def prevA():
                        g_wait(rowsA, gsA)
                        s_issue(j - 1, rowsA, ssA)
                    def prevB():
                        g_wait(rowsB, gsB)
                        s_issue(j - 1, rowsB, ssB)
                    with_parity(jnp.logical_not(even), prevA, prevB)
                return 0

            stripe = segs * SEG
            base_e = sid * stripe
            hi = lo + np_t

            def st_issue(sgi, ssrc, sdst, st_p):
                sbase = base_e + sgi * SEG
                pltpu.async_copy(e_h.at[0, pl.ds(sbase, SEG)], ssrc, st_p)
                pltpu.async_copy(e_h.at[1, pl.ds(sbase, SEG)], sdst, st_p)

            def st_wait(ssrc, sdst, st_p):
                pltpu.make_async_copy(e_h.at[0, pl.ds(0, SEG)], ssrc,
                                      st_p).wait()
                pltpu.make_async_copy(e_h.at[1, pl.ds(0, SEG)], sdst,
                                      st_p).wait()

            st_issue(0, seg_srcA, seg_dstA, stA)

            def seg_body(sgi, carry):
                pos0, bfl = carry
                sbase = base_e + sgi * SEG
                seven = lax.bitwise_and(sgi, 1) == 0
                with_parity(seven,
                            lambda: st_wait(seg_srcA, seg_dstA, stA),
                            lambda: st_wait(seg_srcB, seg_dstB, stB))

                @pl.when(sgi + 1 < segs)
                def _():
                    with_parity(
                        seven,
                        lambda: st_issue(sgi + 1, seg_srcB, seg_dstB, stB),
                        lambda: st_issue(sgi + 1, seg_srcA, seg_dstA, stA))

                def cbody(i, off):
                    s = jnp.where(seven, seg_srcA[pl.ds(i * L, L)],
                                  seg_srcB[pl.ds(i * L, L)])
                    d = jnp.where(seven, seg_dstA[pl.ds(i * L, L)],
                                  seg_dstB[pl.ds(i * L, L)])
                    geid = sbase + i * L + iota
                    m = (geid < E) & (d >= lo) & (d < hi)
                    mi = jnp.where(m, 1, 0).astype(i32)
                    pos = off + plsc.cumsum(mi) - 1
                    posr = lax.bitwise_and(pos, CE - 1)
                    plsc.store_scatter(csrc, [posr], s + b * nsrc, mask=m)
                    plsc.store_scatter(
                        cdst,
                        [lax.shift_right_logical(posr, 6),
                         lax.bitwise_and(posr, G - 1)],
                        d - lo, mask=m)
                    return off + plsc.all_reduce_population_count(m)

                pos1v = lax.fori_loop(0, SEG // L, cbody, pos0, unroll=2)
                pos1 = jnp.max(pos1v)
                bhi = lax.shift_right_logical(pos1, 6)
                lax.fori_loop(bfl, bhi, fbody, 0)
                return pos1v, bhi

            posv, bfl = lax.fori_loop(
                0, segs, seg_body, (jnp.zeros((L,), i32), jnp.int32(0)))
            pos = jnp.max(posv)

            nblk = lax.shift_right_logical(pos + (G - 1), 6)
            lastb = (nblk - 1) * G

            def nb(i, _):
                posv = lastb + i * L + iota
                mm = posv >= pos
                posr = lax.bitwise_and(posv, CE - 1)
                plsc.store_scatter(csrc, [posr], iota, mask=mm)
                plsc.store_scatter(
                    cdst,
                    [lax.shift_right_logical(posr, 6),
                     lax.bitwise_and(posr, G - 1)],
                    TRASH + iota, mask=mm)
                return 0

            lax.fori_loop(0, G // L, nb, 0)
            lax.fori_loop(bfl, nblk, fbody, 0)
            lastev = lax.bitwise_and(nblk - 1, 1) == 0

            @pl.when(nblk >= 1)
            def _():
                def lastA():
                    g_wait(rowsA, gsA)
                    s_issue(nblk - 1, rowsA, ssA)
                def lastB():
                    g_wait(rowsB, gsB)
                    s_issue(nblk - 1, rowsB, ssB)
                with_parity(lastev, lastA, lastB)

            @pl.when(nblk >= 2)
            def _():
                with_parity(jnp.logical_not(lastev),
                            lambda: s_wait(rowsA, ssA),
                            lambda: s_wait(rowsB, ssB))

            @pl.when(nblk >= 1)
            def _():
                with_parity(lastev, lambda: s_wait(rowsA, ssA),
                            lambda: s_wait(rowsB, ssB))

            plsc.subcore_barrier()

            rpt = np_t // NS
            rb = sid * rpt
            wchunks = _chunks(rpt, G)
            for off, sz in wchunks:
                pltpu.async_copy(
                    acc_sum.at[pl.ds(rb + off, sz)],
                    out_sum.at[pl.ds(out_base + rb + off, sz)], wsem)
            pltpu.sync_copy(acc_cnt.at[pl.ds(rb, rpt)], cvec.at[pl.ds(0, rpt)])

            def ib(i, _):
                c = cvec[pl.ds(i * L, L)]
                ivec[pl.ds(i * L, L)] = 1.0 / jnp.maximum(c, 1.0)
                return 0

            lax.fori_loop(0, rpt // L, ib, 0)
            pltpu.sync_copy(ivec.at[pl.ds(0, rpt)],
                            out_inv.at[pl.ds(out_base + rb, rpt)])
            for off, sz in wchunks:
                pltpu.make_async_copy(
                    acc_sum.at[pl.ds(rb + off, sz)],
                    out_sum.at[pl.ds(out_base + rb + off, sz)], wsem).wait()
            plsc.subcore_barrier()

        run_task(ep_h, E_PIPE, SEGS_PIPE, z1_h, N1, NP1, 0, sum_p, inv_p,
                 b * NP1)
        run_task(e21_h, E_C, SEGS_C, z2_h, N2, NP1, 0, sum_21, inv_21,
                 b * NP1)
        run_task(e12_h, E_C, SEGS_C, z1_h, N1, NP1, 0, sum_12, inv_12,
                 b * NP1)

        def surf_chunk(k, _):
            run_task(es_h, E_SURF, SEGS_SURF, z2_h, N2, NCH, k * NCH,
                     sum_s, inv_s, b * NSURF + k * NCH)
            return 0

        lax.fori_loop(0, NSC, surf_chunk, 0)

    return body(z1f, z2f, eip, eis, e12, e21, zvec_h, ones_h)


def _tc_head(sum_a, inv_a, sum_b, inv_b, z, wla, wlb, wra, wrb, wp,
             bla, blb, bp, g, bln, nb_b):
    Bz, NZ, _ = z.shape
    NB = (NZ + RTC - 1) // RTC
    f32 = jnp.float32

    def body(sa, ia, sb, ib_, zz, rwla, rwlb, rwra, rwrb, rwp, rba, rbb,
             rbp, rg, rbl, o):
        i = pl.program_id(1)
        agg_a = sa[0] * ia[0, 0, :][:, None]
        agg_b = sb[0] * ib_[0, 0, :][:, None]
        agg_b = jnp.where(i < nb_b, agg_b, 0.0)
        h = (jnp.dot(agg_a, rwla[...], preferred_element_type=f32)
             + jnp.dot(agg_b, rwlb[...], preferred_element_type=f32)
             + jnp.dot(zz[0], rwra[...] + rwrb[...],
                       preferred_element_type=f32)
             + rba[0] + rbb[0])
        y = jnp.dot(h, rwp[...], preferred_element_type=f32) + rbp[0]
        m = jnp.mean(y, axis=-1, keepdims=True)
        yc = y - m
        v = jnp.mean(yc * yc, axis=-1, keepdims=True)
        o[0] = yc * lax.rsqrt(v + 1e-5) * rg[0] + rbl[0]

    in_specs = [
        pl.BlockSpec((1, RTC, D), lambda b, i: (b, i, 0)),
        pl.BlockSpec((1, 1, RTC), lambda b, i: (b, 0, i)),
        pl.BlockSpec((1, RTC, D), lambda b, i: (b, jnp.minimum(i, nb_b - 1), 0)),
        pl.BlockSpec((1, 1, RTC), lambda b, i: (b, 0, jnp.minimum(i, nb_b - 1))),
        pl.BlockSpec((1, RTC, D), lambda b, i: (b, i, 0)),
    ] + [pl.BlockSpec((D, D), lambda b, i: (0, 0))] * 5 \
      + [pl.BlockSpec((1, D), lambda b, i: (0, 0))] * 5
    return pl.pallas_call(
        body,
        grid=(Bz, NB),
        in_specs=in_specs,
        out_specs=pl.BlockSpec((1, RTC, D), lambda b, i: (b, i, 0)),
        out_shape=jax.ShapeDtypeStruct((Bz, NZ, D), f32),
    )(sum_a, inv_a, sum_b, inv_b, z, wla, wlb, wra, wrb, wp,
      bla, blb, bp, g, bln)


def kernel(z_1d, z_2d, edge_index_pipe, edge_index_surface, edge_index_c12,
           edge_index_c21, params):
    f32, i32 = jnp.float32, jnp.int32
    z1f = z_1d.reshape(B * N1, D)
    z2f = z_2d.reshape(B * N2, D)

    def prep(ei, segs):
        pe = NS * segs * SEG
        ei = ei.astype(i32)
        return jnp.pad(ei, ((0, 0), (0, pe - ei.shape[1])))

    eip = prep(edge_index_pipe, SEGS_PIPE)
    eis = prep(edge_index_surface, SEGS_SURF)
    e12 = prep(edge_index_c12, SEGS_C)
    e21 = prep(edge_index_c21, SEGS_C)
    zvec_h = jnp.zeros((ZPT,), f32)
    ones_h = jnp.ones((G,), f32)

    (sum_p, inv_p, sum_s, inv_s, sum_12, inv_12, sum_21, inv_21) = \
        _sc_segment_sums(z1f, z2f, eip, eis, e12, e21, zvec_h, ones_h)

    sum_p = sum_p.reshape(B, NP1, D)
    sum_s = sum_s.reshape(B, NSURF, D)
    sum_12 = sum_12.reshape(B, NP1, D)
    sum_21 = sum_21.reshape(B, NP1, D)
    inv_p = inv_p.reshape(B, 1, NP1)
    inv_s = inv_s.reshape(B, 1, NSURF)
    inv_12 = inv_12.reshape(B, 1, NP1)
    inv_21 = inv_21.reshape(B, 1, NP1)

    p = params
    r2 = lambda v: v.reshape(1, D)

    d1 = _tc_head(sum_p, inv_p, sum_21, inv_21, z_1d,
                  p['pipe']['Wl'], p['c21']['Wl'],
                  p['pipe']['Wr'], p['c21']['Wr'], p['proj_1d']['W'],
                  r2(p['pipe']['bl']), r2(p['c21']['bl']),
                  r2(p['proj_1d']['b']), r2(p['ln_1d']['g']),
                  r2(p['ln_1d']['b']), NP1 // RTC)
    d2 = _tc_head(sum_s, inv_s, sum_12, inv_12, z_2d,
                  p['surface']['Wl'], p['c12']['Wl'],
                  p['surface']['Wr'], p['c12']['Wr'], p['proj_2d']['W'],
                  r2(p['surface']['bl']), r2(p['c12']['bl']),
                  r2(p['proj_2d']['b']), r2(p['ln_2d']['g']),
                  r2(p['ln_2d']['b']), NP1 // RTC)
    return d1, d2


# speedup vs baseline: 1.0766x; 1.0649x over previous
import functools

import jax
import jax.numpy as jnp
from jax import lax
from jax.experimental import pallas as pl
from jax.experimental.pallas import tpu as pltpu
from jax.experimental.pallas import tpu_sc as plsc

B, N1, N2, D = 2, 10000, 50000, 128
E_PIPE, E_SURF, E_C = 160000, 600000, 40000

NC, NS, L = 2, 16, 16
SEG = 1792
G = 32
GSH = 5
NBUF = 4
CE = 2048
CBLK = CE // G
NP1 = 10240
NCH = 10240
NSC = 5
NSURF = NSC * NCH
ACC_ROWS = 10496
ZPT = ACC_ROWS // NS
TRASH = NCH
SEGS_PIPE, SEGS_SURF, SEGS_C = 6, 21, 2
RTC = 512


def _chunks(total, step):
    out, off = [], 0
    while off < total:
        sz = min(step, total - off)
        out.append((off, sz))
        off += sz
    return out


def _sc_segment_sums(z1f, z2f, eip, eis, e12, e21, zvec_h, ones_h):
    mesh = plsc.VectorSubcoreMesh(core_axis_name="c", subcore_axis_name="s",
                                  num_cores=NC, num_subcores=NS)
    f32, i32 = jnp.float32, jnp.int32
    out_type = [
        jax.ShapeDtypeStruct((B * NP1, D), f32),
        jax.ShapeDtypeStruct((B * NP1,), f32),
        jax.ShapeDtypeStruct((B * NSURF, D), f32),
        jax.ShapeDtypeStruct((B * NSURF,), f32),
        jax.ShapeDtypeStruct((B * NP1, D), f32),
        jax.ShapeDtypeStruct((B * NP1,), f32),
        jax.ShapeDtypeStruct((B * NP1, D), f32),
        jax.ShapeDtypeStruct((B * NP1,), f32),
    ]
    scratch = [
        pltpu.VMEM_SHARED((ACC_ROWS, D), f32),
        pltpu.VMEM_SHARED((ACC_ROWS,), f32),
        pltpu.VMEM((SEG,), i32),
        pltpu.VMEM((SEG,), i32),
        pltpu.VMEM((SEG,), i32),
        pltpu.VMEM((SEG,), i32),
        pltpu.VMEM((CE,), i32),
        pltpu.VMEM((CBLK, G), i32),
    ] + [pltpu.VMEM((G, D), f32)] * NBUF \
      + [
        pltpu.VMEM((ZPT,), f32),
        pltpu.VMEM((G,), f32),
        pltpu.VMEM((NCH // NS,), f32),
    ] + [pltpu.SemaphoreType.DMA] * (2 * NBUF) \
      + [
        pltpu.SemaphoreType.DMA,
        pltpu.SemaphoreType.DMA,
        pltpu.SemaphoreType.DMA,
    ]

    @functools.partial(
        pl.kernel, out_type=out_type, mesh=mesh, scratch_types=scratch,
        compiler_params=pltpu.CompilerParams(needs_layout_passes=False))
    def body(z1_h, z2_h, ep_h, es_h, e12_h, e21_h, zv_h, on_h,
             sum_p, inv_p, sum_s, inv_s, sum_12, inv_12, sum_21, inv_21,
             acc_sum, acc_cnt, seg_srcA, seg_dstA, seg_srcB, seg_dstB,
             csrc, cdst, rows0, rows1, rows2, rows3,
             zvec, onesv, cvec, gs0, gs1, gs2, gs3,
             ss0, ss1, ss2, ss3, stA, stB, wsem):
        rows_b = [rows0, rows1, rows2, rows3]
        gs_b = [gs0, gs1, gs2, gs3]
        ss_b = [ss0, ss1, ss2, ss3]
        rowsA = rows0
        b = lax.axis_index("c")
        sid = lax.axis_index("s")
        iota = lax.broadcasted_iota(i32, (L,), 0)
        pltpu.sync_copy(zv_h, zvec)
        pltpu.sync_copy(on_h, onesv)

        def zero_rows():
            def zb(r, _):
                for c in range(D // L):
                    rowsA[r, pl.ds(c * L, L)] = jnp.zeros((L,), f32)
                return 0
            lax.fori_loop(0, G, zb, 0)

        def with_parity(even, fa, fb):
            @pl.when(even)
            def _():
                fa()
            @pl.when(jnp.logical_not(even))
            def _():
                fb()

        def run_task(e_h, E, segs, z_h, nsrc, np_t, lo, out_sum, out_inv,
                     out_base):
            zero_rows()
            zb = sid * ZPT
            zchunks = _chunks(ZPT, G)
            for off, sz in zchunks:
                pltpu.async_copy(rowsA.at[pl.ds(0, sz)],
                                 acc_sum.at[pl.ds(zb + off, sz)], wsem)
            pltpu.async_copy(zvec, acc_cnt.at[pl.ds(zb, ZPT)], wsem)
            for off, sz in zchunks:
                pltpu.make_async_copy(rowsA.at[pl.ds(0, sz)],
                                      acc_sum.at[pl.ds(zb + off, sz)],
                                      wsem).wait()
            pltpu.make_async_copy(zvec, acc_cnt.at[pl.ds(zb, ZPT)],
                                  wsem).wait()
            plsc.subcore_barrier()

            def g_issue(j, rows_p, gs_p):
                slot = lax.bitwise_and(j, CBLK - 1)
                pltpu.async_copy(z_h.at[csrc.at[pl.ds(slot * G, G)]],
                                 rows_p, gs_p)

            def g_wait(rows_p, gs_p):
                pltpu.make_async_copy(z_h.at[csrc.at[pl.ds(0, G)]],
                                      rows_p, gs_p).wait()

            def s_issue(j, rows_p, ss_p):
                slot = lax.bitwise_and(j, CBLK - 1)
                drow = cdst.at[slot]
                pltpu.async_copy(rows_p, acc_sum.at[drow], ss_p, add=True)
                pltpu.async_copy(onesv, acc_cnt.at[drow], ss_p, add=True)

            def s_wait(rows_p, ss_p):
                pltpu.make_async_copy(rows_p, acc_sum.at[cdst.at[0]],
                                      ss_p).wait()
                pltpu.make_async_copy(onesv, acc_cnt.at[cdst.at[0]],
                                      ss_p).wait()

            def on_buf(sel, fn):
                for p_ in range(NBUF):
                    @pl.when(sel == p_)
                    def _(p_=p_):
                        fn(p_)

            def fbody(j, _):
                @pl.when(j >= NBUF)
                def _():
                    on_buf(lax.bitwise_and(j, NBUF - 1),
                           lambda p: s_wait(rows_b[p], ss_b[p]))
                on_buf(lax.bitwise_and(j, NBUF - 1),
                       lambda p: g_issue(j, rows_b[p], gs_b[p]))
                @pl.when(j >= 2)
                def _():
                    def fin(p):
                        g_wait(rows_b[p], gs_b[p])
                        s_issue(j - 2, rows_b[p], ss_b[p])
                    on_buf(lax.bitwise_and(j - 2, NBUF - 1), fin)
                return 0

            stripe = segs * SEG
            base_e = sid * stripe
            hi = lo + np_t

            def st_issue(sgi, ssrc, sdst, st_p):
                sbase = base_e + sgi * SEG
                pltpu.async_copy(e_h.at[0, pl.ds(sbase, SEG)], ssrc, st_p)
                pltpu.async_copy(e_h.at[1, pl.ds(sbase, SEG)], sdst, st_p)

            def st_wait(ssrc, sdst, st_p):
                pltpu.make_async_copy(e_h.at[0, pl.ds(0, SEG)], ssrc,
                                      st_p).wait()
                pltpu.make_async_copy(e_h.at[1, pl.ds(0, SEG)], sdst,
                                      st_p).wait()

            st_issue(0, seg_srcA, seg_dstA, stA)

            def seg_body(sgi, carry):
                pos0, bfl = carry
                sbase = base_e + sgi * SEG
                seven = lax.bitwise_and(sgi, 1) == 0
                with_parity(seven,
                            lambda: st_wait(seg_srcA, seg_dstA, stA),
                            lambda: st_wait(seg_srcB, seg_dstB, stB))

                @pl.when(sgi + 1 < segs)
                def _():
                    with_parity(
                        seven,
                        lambda: st_issue(sgi + 1, seg_srcB, seg_dstB, stB),
                        lambda: st_issue(sgi + 1, seg_srcA, seg_dstA, stA))

                def cbody(i, off):
                    s = jnp.where(seven, seg_srcA[pl.ds(i * L, L)],
                                  seg_srcB[pl.ds(i * L, L)])
                    d = jnp.where(seven, seg_dstA[pl.ds(i * L, L)],
                                  seg_dstB[pl.ds(i * L, L)])
                    geid = sbase + i * L + iota
                    m = (geid < E) & (d >= lo) & (d < hi)
                    mi = jnp.where(m, 1, 0).astype(i32)
                    pos = off + plsc.cumsum(mi) - 1
                    posr = lax.bitwise_and(pos, CE - 1)
                    plsc.store_scatter(csrc, [posr], s + b * nsrc, mask=m)
                    plsc.store_scatter(
                        cdst,
                        [lax.shift_right_logical(posr, GSH),
                         lax.bitwise_and(posr, G - 1)],
                        d - lo, mask=m)
                    return off + plsc.all_reduce_population_count(m)

                pos1v = lax.fori_loop(0, SEG // L, cbody, pos0, unroll=2)
                pos1 = jnp.max(pos1v)
                bhi = lax.shift_right_logical(pos1, GSH)
                lax.fori_loop(bfl, bhi, fbody, 0)
                return pos1v, bhi

            posv, bfl = lax.fori_loop(
                0, segs, seg_body, (jnp.zeros((L,), i32), jnp.int32(0)))
            pos = jnp.max(posv)

            nblk = lax.shift_right_logical(pos + (G - 1), GSH)
            lastb = (nblk - 1) * G

            def nb(i, _):
                posv = lastb + i * L + iota
                mm = posv >= pos
                posr = lax.bitwise_and(posv, CE - 1)
                plsc.store_scatter(csrc, [posr], iota, mask=mm)
                plsc.store_scatter(
                    cdst,
                    [lax.shift_right_logical(posr, GSH),
                     lax.bitwise_and(posr, G - 1)],
                    TRASH + iota, mask=mm)
                return 0

            lax.fori_loop(0, G // L, nb, 0)
            lax.fori_loop(bfl, nblk, fbody, 0)

            for dt in (2, 1):
                t = nblk - dt

                @pl.when(t >= 0)
                def _(t=t):
                    def fin(p):
                        g_wait(rows_b[p], gs_b[p])
                        s_issue(t, rows_b[p], ss_b[p])
                    on_buf(lax.bitwise_and(t, NBUF - 1), fin)

            for dt in (NBUF, NBUF - 1, NBUF - 2, NBUF - 3):
                t = nblk - dt

                @pl.when(t >= 0)
                def _(t=t):
                    on_buf(lax.bitwise_and(t, NBUF - 1),
                           lambda p: s_wait(rows_b[p], ss_b[p]))

            plsc.subcore_barrier()

            rpt = np_t // NS
            rb = sid * rpt
            wchunks = _chunks(rpt, G)
            for off, sz in wchunks:
                pltpu.async_copy(
                    acc_sum.at[pl.ds(rb + off, sz)],
                    out_sum.at[pl.ds(out_base + rb + off, sz)], wsem)
            pltpu.sync_copy(acc_cnt.at[pl.ds(rb, rpt)], cvec.at[pl.ds(0, rpt)])

            def ib(i, _):
                c = cvec[pl.ds(i * L, L)]
                cvec[pl.ds(i * L, L)] = 1.0 / jnp.maximum(c, 1.0)
                return 0

            lax.fori_loop(0, rpt // L, ib, 0)
            pltpu.sync_copy(cvec.at[pl.ds(0, rpt)],
                            out_inv.at[pl.ds(out_base + rb, rpt)])
            for off, sz in wchunks:
                pltpu.make_async_copy(
                    acc_sum.at[pl.ds(rb + off, sz)],
                    out_sum.at[pl.ds(out_base + rb + off, sz)], wsem).wait()
            plsc.subcore_barrier()

        run_task(ep_h, E_PIPE, SEGS_PIPE, z1_h, N1, NP1, 0, sum_p, inv_p,
                 b * NP1)
        run_task(e21_h, E_C, SEGS_C, z2_h, N2, NP1, 0, sum_21, inv_21,
                 b * NP1)
        run_task(e12_h, E_C, SEGS_C, z1_h, N1, NP1, 0, sum_12, inv_12,
                 b * NP1)

        def surf_chunk(k, _):
            run_task(es_h, E_SURF, SEGS_SURF, z2_h, N2, NCH, k * NCH,
                     sum_s, inv_s, b * NSURF + k * NCH)
            return 0

        lax.fori_loop(0, NSC, surf_chunk, 0)

    return body(z1f, z2f, eip, eis, e12, e21, zvec_h, ones_h)


def _tc_head(sum_a, inv_a, sum_b, inv_b, z, wla, wlb, wra, wrb, wp,
             bla, blb, bp, g, bln, nb_b):
    Bz, NZ, _ = z.shape
    NB = (NZ + RTC - 1) // RTC
    f32 = jnp.float32

    def body(sa, ia, sb, ib_, zz, rwla, rwlb, rwra, rwrb, rwp, rba, rbb,
             rbp, rg, rbl, o):
        i = pl.program_id(1)
        agg_a = sa[0] * ia[0, 0, :][:, None]
        agg_b = sb[0] * ib_[0, 0, :][:, None]
        agg_b = jnp.where(i < nb_b, agg_b, 0.0)
        h = (jnp.dot(agg_a, rwla[...], preferred_element_type=f32)
             + jnp.dot(agg_b, rwlb[...], preferred_element_type=f32)
             + jnp.dot(zz[0], rwra[...] + rwrb[...],
                       preferred_element_type=f32)
             + rba[0] + rbb[0])
        y = jnp.dot(h, rwp[...], preferred_element_type=f32) + rbp[0]
        m = jnp.mean(y, axis=-1, keepdims=True)
        yc = y - m
        v = jnp.mean(yc * yc, axis=-1, keepdims=True)
        o[0] = yc * lax.rsqrt(v + 1e-5) * rg[0] + rbl[0]

    in_specs = [
        pl.BlockSpec((1, RTC, D), lambda b, i: (b, i, 0)),
        pl.BlockSpec((1, 1, RTC), lambda b, i: (b, 0, i)),
        pl.BlockSpec((1, RTC, D), lambda b, i: (b, jnp.minimum(i, nb_b - 1), 0)),
        pl.BlockSpec((1, 1, RTC), lambda b, i: (b, 0, jnp.minimum(i, nb_b - 1))),
        pl.BlockSpec((1, RTC, D), lambda b, i: (b, i, 0)),
    ] + [pl.BlockSpec((D, D), lambda b, i: (0, 0))] * 5 \
      + [pl.BlockSpec((1, D), lambda b, i: (0, 0))] * 5
    return pl.pallas_call(
        body,
        grid=(Bz, NB),
        in_specs=in_specs,
        out_specs=pl.BlockSpec((1, RTC, D), lambda b, i: (b, i, 0)),
        out_shape=jax.ShapeDtypeStruct((Bz, NZ, D), f32),
    )(sum_a, inv_a, sum_b, inv_b, z, wla, wlb, wra, wrb, wp,
      bla, blb, bp, g, bln)


def kernel(z_1d, z_2d, edge_index_pipe, edge_index_surface, edge_index_c12,
           edge_index_c21, params):
    f32, i32 = jnp.float32, jnp.int32
    z1f = z_1d.reshape(B * N1, D)
    z2f = z_2d.reshape(B * N2, D)

    def prep(ei, segs):
        pe = NS * segs * SEG
        ei = ei.astype(i32)
        return jnp.pad(ei, ((0, 0), (0, pe - ei.shape[1])))

    eip = prep(edge_index_pipe, SEGS_PIPE)
    eis = prep(edge_index_surface, SEGS_SURF)
    e12 = prep(edge_index_c12, SEGS_C)
    e21 = prep(edge_index_c21, SEGS_C)
    zvec_h = jnp.zeros((ZPT,), f32)
    ones_h = jnp.ones((G,), f32)

    (sum_p, inv_p, sum_s, inv_s, sum_12, inv_12, sum_21, inv_21) = \
        _sc_segment_sums(z1f, z2f, eip, eis, e12, e21, zvec_h, ones_h)

    sum_p = sum_p.reshape(B, NP1, D)
    sum_s = sum_s.reshape(B, NSURF, D)
    sum_12 = sum_12.reshape(B, NP1, D)
    sum_21 = sum_21.reshape(B, NP1, D)
    inv_p = inv_p.reshape(B, 1, NP1)
    inv_s = inv_s.reshape(B, 1, NSURF)
    inv_12 = inv_12.reshape(B, 1, NP1)
    inv_21 = inv_21.reshape(B, 1, NP1)

    p = params
    r2 = lambda v: v.reshape(1, D)

    d1 = _tc_head(sum_p, inv_p, sum_21, inv_21, z_1d,
                  p['pipe']['Wl'], p['c21']['Wl'],
                  p['pipe']['Wr'], p['c21']['Wr'], p['proj_1d']['W'],
                  r2(p['pipe']['bl']), r2(p['c21']['bl']),
                  r2(p['proj_1d']['b']), r2(p['ln_1d']['g']),
                  r2(p['ln_1d']['b']), NP1 // RTC)
    d2 = _tc_head(sum_s, inv_s, sum_12, inv_12, z_2d,
                  p['surface']['Wl'], p['c12']['Wl'],
                  p['surface']['Wr'], p['c12']['Wr'], p['proj_2d']['W'],
                  r2(p['surface']['bl']), r2(p['c12']['bl']),
                  r2(p['proj_2d']['b']), r2(p['ln_2d']['g']),
                  r2(p['ln_2d']['b']), NP1 // RTC)
    return d1, d2
